# baseline (speedup 1.0000x reference)
import jax
import jax.numpy as jnp
from jax.experimental import pallas as pl
from jax.experimental.pallas import tpu as pltpu

DIM_ = 512
CDIM_ = 13
NQ_ = 8
C_ = 1024
R_ = C_ // 8
PD_ = 8 * DIM_
NCH_ = 8


def _lfq_body(x_hbm, win_hbm, bin_hbm, wout_hbm, bout_hbm, out_hbm, idx_hbm,
              xb, ob, ib, wb_in, bb_in, wb_out, bb_out, w2_ref, w2o_ref,
              insem, outsem, idxsem, wsem):
    cp_w = [pltpu.make_async_copy(s, d, wsem.at[i]) for i, (s, d) in enumerate(
        [(win_hbm, wb_in), (bin_hbm, bb_in), (wout_hbm, wb_out),
         (bout_hbm, bb_out)])]

    def in_copy(c):
        return pltpu.make_async_copy(
            x_hbm.at[pl.ds(c * R_, R_), :], xb.at[c % 3], insem.at[c % 3])

    def out_copy(c):
        return pltpu.make_async_copy(
            ob.at[c % 2], out_hbm.at[pl.ds(c * R_, R_), :], outsem.at[c % 2])

    def idx_copy(c):
        return pltpu.make_async_copy(
            ib.at[c % 2], idx_hbm.at[pl.ds(c * R_, R_), :], idxsem.at[c % 2])

    for cp in cp_w:
        cp.start()
    in_copy(0).start()
    in_copy(1).start()
    for cp in cp_w:
        cp.wait()
    w_in = wb_in[...]
    b2 = bb_in[...]
    w_outT = wb_out[...]
    bout_p = bb_out[...]

    w2_ref[...] = jnp.zeros((PD_, 2 * 64), jnp.float32)
    w2o_ref[...] = jnp.zeros((2 * 64, PD_), jnp.float32)
    w_inT = w_in.T
    for k in range(8):
        w2_ref[pl.ds(512 * k, 512), pl.ds(16 * k, CDIM_)] = w_inT
        w2o_ref[pl.ds(16 * k, CDIM_), pl.ds(512 * k, 512)] = w_outT
    w2 = w2_ref[...]
    w2o = w2o_ref[...]

    row = jax.lax.broadcasted_iota(jnp.int32, (NQ_ * 128, 64), 0)
    col = jax.lax.broadcasted_iota(jnp.int32, (NQ_ * 128, 64), 1)
    i_ = row // 128
    k_ = (row % 128) // 16
    j_ = row % 16
    G = jnp.where((col == 8 * k_ + i_) & (j_ < CDIM_),
                  jnp.exp2(j_.astype(jnp.float32)), 0.0)

    for c in range(NCH_):
        in_copy(c).wait()
        if c + 2 < NCH_:
            in_copy(c + 2).start()
        if c >= 2:
            out_copy(c - 2).wait()
            idx_copy(c - 2).wait()

        x = xb[c % 3]
        h = jax.lax.dot_general(
            x, w2, (((1,), (0,)), ((), ())),
            preferred_element_type=jnp.float32)
        h = h + b2

        r = h
        planes = []
        for i in range(NQ_):
            s = float(2.0 ** (-i))
            bits = r > 0
            bf = jnp.where(bits, 1.0, 0.0)
            r = r - jnp.where(bits, s, -s)
            planes.append(bf)
        bits_all = jnp.concatenate(planes, axis=1)
        idxf = jax.lax.dot_general(
            bits_all, G, (((1,), (0,)), ((), ())),
            preferred_element_type=jnp.float32)
        ib[c % 2] = idxf.astype(jnp.int32)

        q = h - r
        out = jax.lax.dot_general(
            q, w2o, (((1,), (0,)), ((), ())),
            preferred_element_type=jnp.float32)
        ob[c % 2] = out + bout_p

        out_copy(c).start()
        idx_copy(c).start()
    out_copy(NCH_ - 2).wait()
    idx_copy(NCH_ - 2).wait()
    out_copy(NCH_ - 1).wait()
    idx_copy(NCH_ - 1).wait()


def kernel(x, W_in, b_in, W_out, b_out):
    B, N, D = x.shape
    M = B * N
    MP = M // 8
    xm = x.reshape(MP, PD_)
    b2 = jnp.zeros((8, 16), jnp.float32).at[:, :CDIM_].set(
        jnp.broadcast_to(b_in, (8, CDIM_))).reshape(1, 128)
    woutT = W_out.T
    bout2 = jnp.tile(b_out.reshape(1, D), (1, 8))
    out, idx = pl.pallas_call(
        _lfq_body,
        in_specs=[pl.BlockSpec(memory_space=pl.ANY)] * 5,
        out_specs=[pl.BlockSpec(memory_space=pl.ANY)] * 2,
        out_shape=[
            jax.ShapeDtypeStruct((MP, PD_), jnp.float32),
            jax.ShapeDtypeStruct((MP, 64), jnp.int32),
        ],
        scratch_shapes=[
            pltpu.VMEM((3, R_, PD_), jnp.float32),
            pltpu.VMEM((2, R_, PD_), jnp.float32),
            pltpu.VMEM((2, R_, 64), jnp.int32),
            pltpu.VMEM((CDIM_, DIM_), jnp.float32),
            pltpu.VMEM((1, 128), jnp.float32),
            pltpu.VMEM((CDIM_, DIM_), jnp.float32),
            pltpu.VMEM((1, PD_), jnp.float32),
            pltpu.VMEM((PD_, 128), jnp.float32),
            pltpu.VMEM((128, PD_), jnp.float32),
            pltpu.SemaphoreType.DMA((3,)),
            pltpu.SemaphoreType.DMA((2,)),
            pltpu.SemaphoreType.DMA((2,)),
            pltpu.SemaphoreType.DMA((4,)),
        ],
    )(xm, W_in, b2, woutT, bout2)
    losses = jnp.zeros((NQ_,), x.dtype)
    return out.reshape(B, N, D), idx.reshape(B, N, NQ_), losses


# speedup vs baseline: 3.3944x; 3.3944x over previous
import jax
import jax.numpy as jnp
from jax.experimental import pallas as pl
from jax.experimental.pallas import tpu as pltpu

DIM_ = 512
CDIM_ = 13
NQ_ = 8
C_ = 1024
NCH_ = 8


def _lfq_body(x_hbm, win_hbm, bin_hbm, wout_hbm, bout_hbm, out_hbm, idx_hbm,
              xb, ob, ib, wb_in, bb_in, wb_out, bb_out,
              insem, outsem, idxsem, wsem):
    cp_w = [pltpu.make_async_copy(s, d, wsem.at[i]) for i, (s, d) in enumerate(
        [(win_hbm, wb_in), (bin_hbm, bb_in), (wout_hbm, wb_out),
         (bout_hbm, bb_out)])]

    def in_copy(c):
        return pltpu.make_async_copy(
            x_hbm.at[pl.ds(c * C_, C_), :], xb.at[c % 3], insem.at[c % 3])

    def out_copy(c):
        return pltpu.make_async_copy(
            ob.at[c % 2], out_hbm.at[pl.ds(c * C_, C_), :], outsem.at[c % 2])

    def idx_copy(c):
        return pltpu.make_async_copy(
            ib.at[c % 2], idx_hbm.at[pl.ds(c * C_, C_), :], idxsem.at[c % 2])

    for cp in cp_w:
        cp.start()
    in_copy(0).start()
    in_copy(1).start()
    for cp in cp_w:
        cp.wait()
    w_in = wb_in[...]
    b_in = bb_in[...].T
    w_outT = wb_out[...]
    b_out = bb_out[...]

    pow2 = jax.lax.broadcasted_iota(jnp.int32, (CDIM_, 1), 0)
    pow2 = jnp.exp2(pow2.astype(jnp.float32))

    for c in range(NCH_):
        in_copy(c).wait()
        if c + 2 < NCH_:
            in_copy(c + 2).start()
        if c >= 2:
            out_copy(c - 2).wait()
            idx_copy(c - 2).wait()

        x = xb[c % 3]
        h = jax.lax.dot_general(
            w_in, x, (((1,), (1,)), ((), ())),
            preferred_element_type=jnp.float32)
        h = h + b_in

        r = h
        idx_rows = []
        for i in range(NQ_):
            s = float(2.0 ** (-i))
            bits = r > 0
            r = r - jnp.where(bits, s, -s)
            idx_rows.append(
                jnp.sum(jnp.where(bits, pow2, 0.0), axis=0, keepdims=True))
        idx_t = jnp.concatenate(idx_rows, axis=0)
        ib[c % 2] = idx_t.T.astype(jnp.int32)

        q = (h - r).T
        out = jax.lax.dot_general(
            q, w_outT, (((1,), (0,)), ((), ())),
            preferred_element_type=jnp.float32)
        ob[c % 2] = out + b_out

        out_copy(c).start()
        idx_copy(c).start()
    out_copy(NCH_ - 2).wait()
    idx_copy(NCH_ - 2).wait()
    out_copy(NCH_ - 1).wait()
    idx_copy(NCH_ - 1).wait()


def kernel(x, W_in, b_in, W_out, b_out):
    B, N, D = x.shape
    M = B * N
    xm = x.reshape(M, D)
    bin2 = b_in.reshape(1, CDIM_)
    woutT = W_out.T
    bout2 = b_out.reshape(1, D)
    out, idx = pl.pallas_call(
        _lfq_body,
        in_specs=[pl.BlockSpec(memory_space=pl.ANY)] * 5,
        out_specs=[pl.BlockSpec(memory_space=pl.ANY)] * 2,
        out_shape=[
            jax.ShapeDtypeStruct((M, D), jnp.float32),
            jax.ShapeDtypeStruct((M, NQ_), jnp.int32),
        ],
        scratch_shapes=[
            pltpu.VMEM((3, C_, DIM_), jnp.float32),
            pltpu.VMEM((2, C_, DIM_), jnp.float32),
            pltpu.VMEM((2, C_, NQ_), jnp.int32),
            pltpu.VMEM((CDIM_, DIM_), jnp.float32),
            pltpu.VMEM((1, CDIM_), jnp.float32),
            pltpu.VMEM((CDIM_, DIM_), jnp.float32),
            pltpu.VMEM((1, DIM_), jnp.float32),
            pltpu.SemaphoreType.DMA((3,)),
            pltpu.SemaphoreType.DMA((2,)),
            pltpu.SemaphoreType.DMA((2,)),
            pltpu.SemaphoreType.DMA((4,)),
        ],
    )(xm, W_in, bin2, woutT, bout2)
    losses = jnp.zeros((NQ_,), x.dtype)
    return out.reshape(B, N, D), idx.reshape(B, N, NQ_), losses


# speedup vs baseline: 3.7828x; 1.1144x over previous
import jax
import jax.numpy as jnp
from jax.experimental import pallas as pl
from jax.experimental.pallas import tpu as pltpu

DIM_ = 512
CDIM_ = 13
NQ_ = 8
C_ = 2048
NCH_ = 4


def _lfq_body(x_hbm, win_hbm, bin_hbm, wout_hbm, bout_hbm, out_hbm, idx_hbm,
              xb, ob, ib, wb_in, bb_in, wb_out, bb_out,
              insem, outsem, idxsem, wsem):
    cp_w = [pltpu.make_async_copy(s, d, wsem.at[i]) for i, (s, d) in enumerate(
        [(win_hbm, wb_in), (bin_hbm, bb_in), (wout_hbm, wb_out),
         (bout_hbm, bb_out)])]

    def in_copy(c):
        return pltpu.make_async_copy(
            x_hbm.at[pl.ds(c * C_, C_), :], xb.at[c % 3], insem.at[c % 3])

    def out_copy(c):
        return pltpu.make_async_copy(
            ob.at[c % 2], out_hbm.at[pl.ds(c * C_, C_), :], outsem.at[c % 2])

    def idx_copy(c):
        return pltpu.make_async_copy(
            ib.at[c % 2], idx_hbm.at[pl.ds(c * C_, C_), :], idxsem.at[c % 2])

    for cp in cp_w:
        cp.start()
    in_copy(0).start()
    in_copy(1).start()
    for cp in cp_w:
        cp.wait()
    w_in = wb_in[...]
    b_in = bb_in[...].T
    w_outT = wb_out[...]
    b_out = bb_out[...]

    pow2 = jax.lax.broadcasted_iota(jnp.int32, (CDIM_, 1), 0)
    pow2 = jnp.exp2(pow2.astype(jnp.float32))

    for c in range(NCH_):
        in_copy(c).wait()
        if c + 2 < NCH_:
            in_copy(c + 2).start()
        if c >= 2:
            out_copy(c - 2).wait()
            idx_copy(c - 2).wait()

        x = xb[c % 3]
        h = jax.lax.dot_general(
            w_in, x, (((1,), (1,)), ((), ())),
            preferred_element_type=jnp.float32)
        h = h + b_in

        r = h
        idx_rows = []
        for i in range(NQ_):
            s = float(2.0 ** (-i))
            bits = r > 0
            r = r - jnp.where(bits, s, -s)
            idx_rows.append(
                jnp.sum(jnp.where(bits, pow2, 0.0), axis=0, keepdims=True))
        idx_t = jnp.concatenate(idx_rows, axis=0)
        ib[c % 2] = idx_t.T.astype(jnp.int32)

        q = (h - r).T
        out = jax.lax.dot_general(
            q, w_outT, (((1,), (0,)), ((), ())),
            preferred_element_type=jnp.float32)
        ob[c % 2] = out + b_out

        out_copy(c).start()
        idx_copy(c).start()
    out_copy(NCH_ - 2).wait()
    idx_copy(NCH_ - 2).wait()
    out_copy(NCH_ - 1).wait()
    idx_copy(NCH_ - 1).wait()


def kernel(x, W_in, b_in, W_out, b_out):
    B, N, D = x.shape
    M = B * N
    xm = x.reshape(M, D)
    bin2 = b_in.reshape(1, CDIM_)
    woutT = W_out.T
    bout2 = b_out.reshape(1, D)
    out, idx = pl.pallas_call(
        _lfq_body,
        in_specs=[pl.BlockSpec(memory_space=pl.ANY)] * 5,
        out_specs=[pl.BlockSpec(memory_space=pl.ANY)] * 2,
        out_shape=[
            jax.ShapeDtypeStruct((M, D), jnp.float32),
            jax.ShapeDtypeStruct((M, NQ_), jnp.int32),
        ],
        scratch_shapes=[
            pltpu.VMEM((3, C_, DIM_), jnp.float32),
            pltpu.VMEM((2, C_, DIM_), jnp.float32),
            pltpu.VMEM((2, C_, NQ_), jnp.int32),
            pltpu.VMEM((CDIM_, DIM_), jnp.float32),
            pltpu.VMEM((1, CDIM_), jnp.float32),
            pltpu.VMEM((CDIM_, DIM_), jnp.float32),
            pltpu.VMEM((1, DIM_), jnp.float32),
            pltpu.SemaphoreType.DMA((3,)),
            pltpu.SemaphoreType.DMA((2,)),
            pltpu.SemaphoreType.DMA((2,)),
            pltpu.SemaphoreType.DMA((4,)),
        ],
    )(xm, W_in, bin2, woutT, bout2)
    losses = jnp.zeros((NQ_,), x.dtype)
    return out.reshape(B, N, D), idx.reshape(B, N, NQ_), losses
